# baseline (speedup 1.0000x reference)
import functools

import jax
import jax.numpy as jnp
from jax import lax
from jax.experimental import pallas as pl
from jax.experimental.pallas import tpu as pltpu
from jax.experimental.pallas import tpu_sc as plsc

D = 64
NG = 16
MP = 16
Q = NG * MP
LMD = 0.5
IDX_CHUNK = 128


def _sc_gather(item_pairs, pair_idx2d, user_embed, B):
    NC, NS = 2, 16
    NW = NC * NS
    b_per_w = B // NW
    n_chunks = b_per_w // IDX_CHUNK
    mesh = plsc.VectorSubcoreMesh(
        core_axis_name="c", subcore_axis_name="s",
        num_cores=NC, num_subcores=NS)

    @functools.partial(
        pl.kernel,
        out_type=jax.ShapeDtypeStruct((B, 2 * D), jnp.float32),
        mesh=mesh,
        scratch_types=[
            pltpu.VMEM((n_chunks, IDX_CHUNK), jnp.int32),
            pltpu.VMEM((b_per_w, 2 * D), jnp.float32),
            pltpu.SemaphoreType.DMA,
        ],
        compiler_params=pltpu.CompilerParams(use_tc_tiling_on_sc=True),
    )
    def gather_k(pair_tbl, iidx, user_tbl, item_out, idx_v, rows_v, sem):
        wid = lax.axis_index("s") * NC + lax.axis_index("c")
        base = wid * b_per_w
        pltpu.sync_copy(iidx.at[pl.ds(wid * n_chunks, n_chunks)], idx_v)
        copies = []
        for c in range(n_chunks):
            copies.append(pltpu.async_copy(
                pair_tbl.at[idx_v.at[c]],
                rows_v.at[pl.ds(c * IDX_CHUNK, IDX_CHUNK)], sem))
        for cp in copies:
            cp.wait()
        pltpu.sync_copy(rows_v, item_out.at[pl.ds(base, b_per_w)])

    return gather_k(item_pairs, pair_idx2d, user_embed)


def _tc_body(item2_ref, ohp_ref, midx_ref, user_hbm, maskmm_ref, maskgm_ref,
             gemb_ref, w1at_ref, w1bt_ref, b1_ref, w2t_ref, b2_ref,
             wpa_ref, wpb_ref, wpc2_ref, pb1_ref, wp2t_ref, pb2_ref,
             out_ref, WC_s, U2_s, W2rep_s, MR_s, sem):

    @pl.when(pl.program_id(0) == 0)
    def _prep():
        def fire(j, carry):
            idx = midx_ref[j]
            pltpu.make_async_copy(
                user_hbm.at[pl.ds(idx, 1)], MR_s.at[pl.ds(j, 1)], sem).start()
            return carry
        lax.fori_loop(0, Q, fire, 0)

        def drain(j, carry):
            idx = midx_ref[j]
            pltpu.make_async_copy(
                user_hbm.at[pl.ds(idx, 1)], MR_s.at[pl.ds(j, 1)], sem).wait()
            return carry
        lax.fori_loop(0, Q, drain, 0)

        U2 = MR_s[...] * maskmm_ref[...]
        U2_s[...] = U2
        w1at = w1at_ref[...]
        qi = lax.broadcasted_iota(jnp.int32, (MP, Q), 1)
        ri = lax.broadcasted_iota(jnp.int32, (MP, Q), 0)
        acc = jnp.zeros((NG, Q), jnp.float32)
        for m in range(MP):
            Pm = U2[m * NG:(m + 1) * NG, :] @ w1at
            Em = ((qi // MP == m) & (qi % MP == ri)).astype(jnp.float32)
            acc = acc + Pm @ Em
        WC_s[0:NG, :] = acc
        WC_s[NG:2 * NG, :] = (qi % MP == ri).astype(jnp.float32)
        q2 = lax.broadcasted_iota(jnp.int32, (Q, MP), 0)
        m2 = lax.broadcasted_iota(jnp.int32, (Q, MP), 1)
        v256 = (q2 % MP == m2).astype(jnp.float32) @ w2t_ref[...]
        W2rep_s[...] = (q2 // MP == m2).astype(jnp.float32) * v256

    ohp = ohp_ref[...]
    oh = ohp[:, :NG]
    par = ohp[:, NG:NG + 1]
    item2 = item2_ref[...]
    itemf = item2[:, :D] + par * (item2[:, D:] - item2[:, :D])
    BB = item2 @ w1bt_ref[...]
    Bmat = BB[:, :NG] + par * (BB[:, NG:] - BB[:, :NG]) + b1_ref[...]

    mq = lax.broadcasted_iota(jnp.int32, (MP, Q), 1)
    mr = lax.broadcasted_iota(jnp.int32, (MP, Q), 0)
    Tmod = (mq % MP == mr).astype(jnp.float32)
    Tdiv = (mq // MP == mr).astype(jnp.float32)

    hfull = jnp.maximum(jnp.concatenate([oh, Bmat], axis=1) @ WC_s[...], 0.0)
    logits = hfull @ W2rep_s[...] + b2_ref[...]
    logits = jnp.clip(logits, -50.0, 50.0)
    w = jnp.exp(logits) * (oh @ maskgm_ref[...])
    w = w / jnp.sum(w, axis=1, keepdims=True)
    wvec = (w @ Tdiv) * (oh @ Tmod)
    g_att = wvec @ U2_s[...]
    g = LMD * g_att + oh @ gemb_ref[...]
    CC = item2 @ wpc2_ref[...]
    itemC = CC[:, :8] + par * (CC[:, 8:] - CC[:, :8])
    hp = jnp.maximum((g * itemf) @ wpa_ref[...] + g @ wpb_ref[...]
                     + itemC + pb1_ref[...], 0.0)
    out_ref[...] = lax.dot_general(
        wp2t_ref[...], hp, (((1,), (1,)), ((), ()))) + pb2_ref[...]


def _tc_main(item2, ohp, member_idx, user_embed, mask_mm, mask_gm,
             gemb, W1aT, W1bT2, b1row, W2T, b2s, WpA, WpB, WpC2, pb1row,
             Wp2row, pb2s, block_b, interpret=False):
    B = item2.shape[0]
    grid = (B // block_b,)
    full = lambda shape: pl.BlockSpec(shape, lambda i: (0, 0))
    return pl.pallas_call(
        _tc_body,
        grid=grid,
        in_specs=[
            pl.BlockSpec((block_b, 2 * D), lambda i: (i, 0)),
            pl.BlockSpec((block_b, 2 * NG), lambda i: (i, 0)),
            pl.BlockSpec(memory_space=pltpu.SMEM),
            pl.BlockSpec(memory_space=pltpu.HBM),
            full((Q, 1)), full((NG, MP)), full((NG, D)),
            full((D, MP)), full((2 * D, 2 * NG)), full((1, MP)),
            full((MP, 1)), full((1, 1)), full((D, 8)), full((D, 8)),
            full((2 * D, 16)), full((1, 8)), full((1, 8)), full((1, 1)),
        ],
        out_specs=pl.BlockSpec((1, block_b), lambda i: (0, i)),
        out_shape=jax.ShapeDtypeStruct((1, B), jnp.float32),
        scratch_shapes=[
            pltpu.VMEM((2 * NG, Q), jnp.float32),
            pltpu.VMEM((Q, D), jnp.float32),
            pltpu.VMEM((Q, MP), jnp.float32),
            pltpu.VMEM((Q, D), jnp.float32),
            pltpu.SemaphoreType.DMA,
        ],
        compiler_params=pltpu.CompilerParams(
            dimension_semantics=("arbitrary",)),
        interpret=interpret,
    )(item2, ohp, member_idx, user_embed, mask_mm, mask_gm, gemb,
      W1aT, W1bT2, b1row, W2T, b2s, WpA, WpB, WpC2, pb1row, Wp2row, pb2s)


def kernel(user_inputs, item_inputs, user_embed, item_embed, group_embed,
           att_W1, att_b1, att_W2, att_b2, pred_W1, pred_b1, pred_W2,
           pred_b2, member_table, member_mask):
    B = user_inputs.shape[0]
    Gn, M = member_table.shape
    tbl = jnp.zeros((NG, MP), jnp.int32).at[:Gn, :M].set(member_table)
    msk = jnp.zeros((NG, MP), jnp.float32).at[:Gn, :M].set(member_mask)
    member_idx = tbl.T.reshape(-1)
    mask_mm = msk.T.reshape(-1, 1)

    item_pairs = item_embed.reshape(-1, 2 * D)
    pair_idx2d = (item_inputs >> 1).reshape(B // IDX_CHUNK, IDX_CHUNK)
    item2 = _sc_gather(item_pairs, pair_idx2d, user_embed, B)

    oh = (user_inputs[:, None] == jnp.arange(NG, dtype=user_inputs.dtype))
    par = (item_inputs[:, None] & 1)
    ohp = jnp.concatenate(
        [oh.astype(jnp.float32), par.astype(jnp.float32),
         jnp.zeros((B, NG - 1), jnp.float32)], axis=1)

    W1aT = att_W1[:, :D].T
    W1bT = att_W1[:, D:].T
    W1bT2 = jnp.zeros((2 * D, 2 * NG), jnp.float32
                      ).at[:D, :NG].set(W1bT).at[D:, NG:].set(W1bT)
    WpC = pred_W1[:, 2 * D:].T
    WpC2 = jnp.zeros((2 * D, 16), jnp.float32
                     ).at[:D, :8].set(WpC).at[D:, 8:].set(WpC)
    preds = _tc_main(
        item2, ohp, member_idx, user_embed, mask_mm, msk, group_embed,
        W1aT, W1bT2, att_b1.reshape(1, MP), att_W2.T, att_b2.reshape(1, 1),
        pred_W1[:, :D].T, pred_W1[:, D:2 * D].T, WpC2,
        pred_b1.reshape(1, 8), pred_W2.reshape(1, 8),
        pred_b2.reshape(1, 1), block_b=4096)
    return preds.reshape(B, 1)


# speedup vs baseline: 1.0445x; 1.0445x over previous
import functools

import jax
import jax.numpy as jnp
from jax import lax
from jax.experimental import pallas as pl
from jax.experimental.pallas import tpu as pltpu
from jax.experimental.pallas import tpu_sc as plsc

D = 64
NG = 16
MP = 16
Q = NG * MP
LMD = 0.5
IDX_CHUNK = 128


def _sc_gather(item_pairs, pair_idx2d, user_embed, B):
    NC, NS = 2, 16
    NW = NC * NS
    b_per_w = B // NW
    n_chunks = b_per_w // IDX_CHUNK
    mesh = plsc.VectorSubcoreMesh(
        core_axis_name="c", subcore_axis_name="s",
        num_cores=NC, num_subcores=NS)

    @functools.partial(
        pl.kernel,
        out_type=jax.ShapeDtypeStruct((B, 2 * D), jnp.float32),
        mesh=mesh,
        scratch_types=[
            pltpu.VMEM((n_chunks, IDX_CHUNK), jnp.int32),
            pltpu.VMEM((b_per_w, 2 * D), jnp.float32),
            pltpu.SemaphoreType.DMA,
        ],
        compiler_params=pltpu.CompilerParams(use_tc_tiling_on_sc=True),
    )
    def gather_k(pair_tbl, iidx, user_tbl, item_out, idx_v, rows_v, sem):
        wid = lax.axis_index("s") * NC + lax.axis_index("c")
        base = wid * b_per_w
        pltpu.sync_copy(iidx.at[pl.ds(wid * n_chunks, n_chunks)], idx_v)
        copies = []
        for c in range(n_chunks):
            copies.append(pltpu.async_copy(
                pair_tbl.at[idx_v.at[c]],
                rows_v.at[pl.ds(c * IDX_CHUNK, IDX_CHUNK)], sem))
        for cp in copies:
            cp.wait()
        pltpu.sync_copy(rows_v, item_out.at[pl.ds(base, b_per_w)])

    return gather_k(item_pairs, pair_idx2d, user_embed)


def _tc_body(item2_ref, ohp_ref, midx_ref, user_hbm, maskmm_ref, maskgm_ref,
             gemb_ref, w1at_ref, w1bt_ref, b1_ref, w2t_ref, b2_ref,
             wp_ref, pb1_ref, wp2t_ref, pb2_ref,
             out_ref, WC_s, U2_s, W2rep_s, MR_s, sem):

    @pl.when(pl.program_id(0) == 0)
    def _prep():
        def fire(j, carry):
            idx = midx_ref[j]
            pltpu.make_async_copy(
                user_hbm.at[pl.ds(idx, 1)], MR_s.at[pl.ds(j, 1)], sem).start()
            return carry
        lax.fori_loop(0, Q, fire, 0)

        def drain(j, carry):
            idx = midx_ref[j]
            pltpu.make_async_copy(
                user_hbm.at[pl.ds(idx, 1)], MR_s.at[pl.ds(j, 1)], sem).wait()
            return carry
        lax.fori_loop(0, Q, drain, 0)

        U2 = MR_s[...] * maskmm_ref[...]
        U2_s[...] = U2
        w1at = w1at_ref[...]
        qi = lax.broadcasted_iota(jnp.int32, (MP, Q), 1)
        ri = lax.broadcasted_iota(jnp.int32, (MP, Q), 0)
        acc = jnp.zeros((NG, Q), jnp.float32)
        for m in range(MP):
            Pm = U2[m * NG:(m + 1) * NG, :] @ w1at
            Em = ((qi // MP == m) & (qi % MP == ri)).astype(jnp.float32)
            acc = acc + Pm @ Em
        WC_s[0:NG, :] = acc
        WC_s[NG:2 * NG, :] = (qi % MP == ri).astype(jnp.float32)
        q2 = lax.broadcasted_iota(jnp.int32, (Q, MP), 0)
        m2 = lax.broadcasted_iota(jnp.int32, (Q, MP), 1)
        v256 = (q2 % MP == m2).astype(jnp.float32) @ w2t_ref[...]
        W2rep_s[...] = (q2 // MP == m2).astype(jnp.float32) * v256

    ohp = ohp_ref[...]
    oh = ohp[:, :NG]
    par = ohp[:, NG:NG + 1]
    item2 = item2_ref[...]
    itemf = item2[:, :D] + par * (item2[:, D:] - item2[:, :D])
    Bmat = itemf @ w1bt_ref[...] + b1_ref[...]

    mq = lax.broadcasted_iota(jnp.int32, (MP, Q), 1)
    mr = lax.broadcasted_iota(jnp.int32, (MP, Q), 0)
    Tmod = (mq % MP == mr).astype(jnp.float32)
    Tdiv = (mq // MP == mr).astype(jnp.float32)

    hfull = jnp.maximum(jnp.concatenate([oh, Bmat], axis=1) @ WC_s[...], 0.0)
    logits = hfull @ W2rep_s[...] + b2_ref[...]
    logits = jnp.clip(logits, -50.0, 50.0)
    w = jnp.exp(logits) * (oh @ maskgm_ref[...])
    w = w / jnp.sum(w, axis=1, keepdims=True)
    wvec = (w @ Tdiv) * (oh @ Tmod)
    g_att = wvec @ U2_s[...]
    g = LMD * g_att + oh @ gemb_ref[...]
    new = jnp.concatenate([g * itemf, g, itemf], axis=1)
    hp = jnp.maximum(new @ wp_ref[...] + pb1_ref[...], 0.0)
    out_ref[...] = lax.dot_general(
        wp2t_ref[...], hp, (((1,), (1,)), ((), ()))) + pb2_ref[...]


def _tc_main(item2, ohp, member_idx, user_embed, mask_mm, mask_gm,
             gemb, W1aT, W1bT, b1row, W2T, b2s, Wp, pb1row, Wp2row, pb2s,
             block_b, interpret=False):
    B = item2.shape[0]
    grid = (B // block_b,)
    full = lambda shape: pl.BlockSpec(shape, lambda i: (0, 0))
    return pl.pallas_call(
        _tc_body,
        grid=grid,
        in_specs=[
            pl.BlockSpec((block_b, 2 * D), lambda i: (i, 0)),
            pl.BlockSpec((block_b, 2 * NG), lambda i: (i, 0)),
            pl.BlockSpec(memory_space=pltpu.SMEM),
            pl.BlockSpec(memory_space=pltpu.HBM),
            full((Q, 1)), full((NG, MP)), full((NG, D)),
            full((D, MP)), full((D, MP)), full((1, MP)), full((MP, 1)),
            full((1, 1)), full((3 * D, 8)),
            full((1, 8)), full((1, 8)), full((1, 1)),
        ],
        out_specs=pl.BlockSpec((1, block_b), lambda i: (0, i)),
        out_shape=jax.ShapeDtypeStruct((1, B), jnp.float32),
        scratch_shapes=[
            pltpu.VMEM((2 * NG, Q), jnp.float32),
            pltpu.VMEM((Q, D), jnp.float32),
            pltpu.VMEM((Q, MP), jnp.float32),
            pltpu.VMEM((Q, D), jnp.float32),
            pltpu.SemaphoreType.DMA,
        ],
        compiler_params=pltpu.CompilerParams(
            dimension_semantics=("arbitrary",)),
        interpret=interpret,
    )(item2, ohp, member_idx, user_embed, mask_mm, mask_gm, gemb,
      W1aT, W1bT, b1row, W2T, b2s, Wp, pb1row, Wp2row, pb2s)


def kernel(user_inputs, item_inputs, user_embed, item_embed, group_embed,
           att_W1, att_b1, att_W2, att_b2, pred_W1, pred_b1, pred_W2,
           pred_b2, member_table, member_mask):
    B = user_inputs.shape[0]
    Gn, M = member_table.shape
    tbl = jnp.zeros((NG, MP), jnp.int32).at[:Gn, :M].set(member_table)
    msk = jnp.zeros((NG, MP), jnp.float32).at[:Gn, :M].set(member_mask)
    member_idx = tbl.T.reshape(-1)
    mask_mm = msk.T.reshape(-1, 1)

    item_pairs = item_embed.reshape(-1, 2 * D)
    pair_idx2d = (item_inputs >> 1).reshape(B // IDX_CHUNK, IDX_CHUNK)
    item2 = _sc_gather(item_pairs, pair_idx2d, user_embed, B)

    oh = (user_inputs[:, None] == jnp.arange(NG, dtype=user_inputs.dtype))
    par = (item_inputs[:, None] & 1)
    ohp = jnp.concatenate(
        [oh.astype(jnp.float32), par.astype(jnp.float32),
         jnp.zeros((B, NG - 1), jnp.float32)], axis=1)

    W1aT = att_W1[:, :D].T
    W1bT = att_W1[:, D:].T
    preds = _tc_main(
        item2, ohp, member_idx, user_embed, mask_mm, msk, group_embed,
        W1aT, W1bT, att_b1.reshape(1, MP), att_W2.T, att_b2.reshape(1, 1),
        pred_W1.T, pred_b1.reshape(1, 8), pred_W2.reshape(1, 8),
        pred_b2.reshape(1, 1), block_b=4096)
    return preds.reshape(B, 1)
